# Initial kernel scaffold; baseline (speedup 1.0000x reference)
import jax
import jax.numpy as jnp
from jax.experimental import pallas as pl


def kernel(x, edge_indexs, tconv_w, tconv_b, w1l, b1l, w1r, w2l, b2l, w2r, wih, whh, bih, bhh):
    raise NotImplementedError("write your pallas kernel here")


# speedup vs baseline: 6.7674x; 6.7674x over previous
import functools

import jax
import jax.numpy as jnp
from jax import lax
from jax.experimental import pallas as pl
from jax.experimental.pallas import tpu as pltpu
from jax.experimental.pallas import tpu_sc as plsc

N_, C_, H_, T_, E_, K_ = 10000, 128, 128, 4, 320000, 3
NCORE, NSUB, LANE = 2, 16, 16
NTILE = NCORE * NSUB
EPT = E_ // NTILE
CH = 80
NCHUNK = EPT // CH
NSC = CH // LANE
ROWB = 624
BN = 1000
_PREC = jax.lax.Precision.HIGHEST


def _segsum_sc(feats, src_e, dst_e, with_count):
    mesh = plsc.VectorSubcoreMesh(core_axis_name="c", subcore_axis_name="s",
                                  num_cores=NCORE, num_subcores=NSUB)
    out_type = [jax.ShapeDtypeStruct((NCORE, N_, C_), jnp.float32)]
    scratch = [
        pltpu.VMEM_SHARED((N_, C_), jnp.float32),
        pltpu.VMEM((EPT,), jnp.int32),
        pltpu.VMEM((EPT,), jnp.int32),
        pltpu.VMEM((CH, C_), jnp.float32),
        pltpu.VMEM((CH, C_), jnp.float32),
        pltpu.SemaphoreType.DMA,
        pltpu.SemaphoreType.DMA,
        pltpu.SemaphoreType.DMA,
        pltpu.SemaphoreType.DMA,
        pltpu.SemaphoreType.DMA,
        pltpu.SemaphoreType.DMA,
    ]
    if with_count:
        out_type.append(jax.ShapeDtypeStruct((NCORE * N_,), jnp.float32))
        scratch += [
            pltpu.VMEM_SHARED((N_,), jnp.float32),
            pltpu.VMEM((CH,), jnp.float32),
            pltpu.VMEM((LANE,), jnp.float32),
            pltpu.VMEM((ROWB,), jnp.float32),
            pltpu.SemaphoreType.DMA,
            pltpu.SemaphoreType.DMA,
        ]

    def body(x_hbm, srce_hbm, dste_hbm, *rest):
        if with_count:
            (out_hbm, cnt_hbm, agg_s, src_v, dst_v, bufA, bufB,
             semA, semB, semZ, semI, semR, semO,
             cnt_s, zc, ones_v, cbuf, semC, semO2) = rest
        else:
            (out_hbm, agg_s, src_v, dst_v, bufA, bufB,
             semA, semB, semZ, semI, semR, semO) = rest
        cid = lax.axis_index("c")
        sid = lax.axis_index("s")
        w = cid * NSUB + sid
        z16 = jnp.zeros((LANE,), jnp.float32)

        def zrow(r, carry):
            for k in range(C_ // LANE):
                bufA[r, pl.ds(k * LANE, LANE)] = z16
            return carry
        lax.fori_loop(0, CH, zrow, 0)
        if with_count:
            for k in range(CH // LANE):
                zc[pl.ds(k * LANE, LANE)] = z16
            ones_v[pl.ds(0, LANE)] = jnp.full((LANE,), 1.0, jnp.float32)

        base = sid * ROWB
        descs = []
        for j in range(7):
            descs.append(pltpu.async_copy(
                bufA.at[pl.ds(0, CH)], agg_s.at[pl.ds(base + j * CH, CH)], semZ))
        descs.append(pltpu.async_copy(
            bufA.at[pl.ds(0, ROWB - 7 * CH)],
            agg_s.at[pl.ds(base + 7 * CH, ROWB - 7 * CH)], semZ))
        if with_count:
            descs.append(pltpu.async_copy(
                zc.at[pl.ds(0, CH)], cnt_s.at[pl.ds(base, CH)], semZ))
            for j in range(1, 7):
                descs.append(pltpu.async_copy(
                    zc.at[pl.ds(0, CH)], cnt_s.at[pl.ds(base + j * CH, CH)], semZ))
            descs.append(pltpu.async_copy(
                zc.at[pl.ds(0, ROWB - 7 * CH)],
                cnt_s.at[pl.ds(base + 7 * CH, ROWB - 7 * CH)], semZ))

        @pl.when(sid == NSUB - 1)
        def _zero_tail():
            tds = [pltpu.async_copy(
                bufA.at[pl.ds(0, N_ - NSUB * ROWB)],
                agg_s.at[pl.ds(NSUB * ROWB, N_ - NSUB * ROWB)], semZ)]
            if with_count:
                tds.append(pltpu.async_copy(
                    zc.at[pl.ds(0, N_ - NSUB * ROWB)],
                    cnt_s.at[pl.ds(NSUB * ROWB, N_ - NSUB * ROWB)], semZ))
            for d in tds:
                d.wait()
        descs.append(pltpu.async_copy(
            srce_hbm.at[pl.ds(w * EPT, EPT)], src_v, semI))
        descs.append(pltpu.async_copy(
            dste_hbm.at[pl.ds(w * EPT, EPT)], dst_v, semI))
        for d in descs:
            d.wait()
        plsc.subcore_barrier()

        def gather(c, buf, sem):
            for k in range(NSC):
                srcv = src_v[pl.ds(c * CH + k * LANE, LANE)]
                pltpu.async_copy(
                    x_hbm.at[srcv], buf.at[pl.ds(k * LANE, LANE)], sem)

        def wait_gather(buf, sem):
            pltpu.make_async_copy(x_hbm.at[pl.ds(0, CH)], buf, sem).wait()

        def scatter_add(c, buf):
            ds = []
            for k in range(NSC):
                dstv = dst_v[pl.ds(c * CH + k * LANE, LANE)]
                ds.append(pltpu.async_copy(
                    buf.at[pl.ds(k * LANE, LANE)], agg_s.at[dstv], semR,
                    add=True))
                if with_count:
                    ds.append(pltpu.async_copy(
                        ones_v, cnt_s.at[dstv], semC, add=True))
            for d in ds:
                d.wait()

        gather(0, bufA, semA)
        gather(1, bufB, semB)

        def loop_body(i, carry):
            cA = 2 * i
            wait_gather(bufA, semA)
            scatter_add(cA, bufA)
            gather(cA + 2, bufA, semA)
            cB = 2 * i + 1
            wait_gather(bufB, semB)
            scatter_add(cB, bufB)
            cB2 = jnp.minimum(cB + 2, NCHUNK - 1)
            gather(cB2, bufB, semB)
            return carry
        lax.fori_loop(0, (NCHUNK - 1) // 2, loop_body, 0)

        cL = NCHUNK - 1
        wait_gather(bufA, semA)
        scatter_add(cL, bufA)
        wait_gather(bufB, semB)

        plsc.subcore_barrier()

        outd = [pltpu.async_copy(
            agg_s.at[pl.ds(base, ROWB)],
            out_hbm.at[cid, pl.ds(base, ROWB)], semO)]
        if with_count:
            pltpu.sync_copy(cnt_s.at[pl.ds(base, ROWB)], cbuf)
            outd.append(pltpu.async_copy(
                cbuf, cnt_hbm.at[pl.ds(cid * N_ + base, ROWB)], semO2))
        for d in outd:
            d.wait()

        @pl.when(sid == NSUB - 1)
        def _read_tail():
            tds = [pltpu.async_copy(
                agg_s.at[pl.ds(NSUB * ROWB, N_ - NSUB * ROWB)],
                out_hbm.at[cid, pl.ds(NSUB * ROWB, N_ - NSUB * ROWB)], semO)]
            if with_count:
                pltpu.sync_copy(cnt_s.at[pl.ds(NSUB * ROWB, N_ - NSUB * ROWB)],
                                cbuf.at[pl.ds(0, N_ - NSUB * ROWB)])
                tds.append(pltpu.async_copy(
                    cbuf.at[pl.ds(0, N_ - NSUB * ROWB)],
                    cnt_hbm.at[pl.ds(cid * N_ + NSUB * ROWB, N_ - NSUB * ROWB)],
                    semO2))
            for d in tds:
                d.wait()

    f = pl.kernel(body, out_type=tuple(out_type), mesh=mesh,
                  scratch_types=tuple(scratch))
    return f(feats, src_e, dst_e)


def _causal_conv(x, wks, b2d):
    def conv_body(x_ref, w_ref, b_ref, o_ref):
        for t in range(T_):
            s = None
            for k in range(K_):
                tt = t + k - (K_ - 1)
                if tt < 0:
                    continue
                m = jnp.dot(x_ref[tt], w_ref[k],
                            preferred_element_type=jnp.float32,
                            precision=_PREC)
                s = m if s is None else s + m
            o_ref[t] = s + b_ref[...]
    return pl.pallas_call(
        conv_body,
        grid=(N_ // BN,),
        in_specs=[
            pl.BlockSpec((T_, BN, C_), lambda i: (0, i, 0)),
            pl.BlockSpec((K_, C_, C_), lambda i: (0, 0, 0)),
            pl.BlockSpec((1, C_), lambda i: (0, 0)),
        ],
        out_specs=pl.BlockSpec((T_, BN, C_), lambda i: (0, i, 0)),
        out_shape=jax.ShapeDtypeStruct((T_, N_, C_), jnp.float32),
    )(x, wks, b2d)


def _sage_dense(agg2, cnt2, xin, wlT, bl2d, wrT, relu):
    def dense_body(a_ref, c_ref, x_ref, wl_ref, b_ref, wr_ref, o_ref):
        agg = a_ref[0] + a_ref[1]
        cnt = c_ref[0, :, 0] + c_ref[1, :, 0]
        mean = agg * (1.0 / jnp.maximum(cnt, 1.0))[:, None]
        y = (jnp.dot(mean, wl_ref[...], preferred_element_type=jnp.float32,
                     precision=_PREC)
             + b_ref[...]
             + jnp.dot(x_ref[...], wr_ref[...],
                       preferred_element_type=jnp.float32, precision=_PREC))
        o_ref[...] = jnp.maximum(y, 0.0) if relu else y
    return pl.pallas_call(
        dense_body,
        grid=(N_ // BN,),
        in_specs=[
            pl.BlockSpec((NCORE, BN, C_), lambda i: (0, i, 0)),
            pl.BlockSpec((NCORE, BN, 1), lambda i: (0, i, 0)),
            pl.BlockSpec((BN, C_), lambda i: (i, 0)),
            pl.BlockSpec((C_, H_), lambda i: (0, 0)),
            pl.BlockSpec((1, H_), lambda i: (0, 0)),
            pl.BlockSpec((C_, H_), lambda i: (0, 0)),
        ],
        out_specs=pl.BlockSpec((BN, H_), lambda i: (i, 0)),
        out_shape=jax.ShapeDtypeStruct((N_, H_), jnp.float32),
    )(agg2, cnt2, xin, wlT, bl2d, wrT)


def _lstm(seq, wihT, whhT, bias2d):
    def lstm_body(s0, s1, s2, s3, wi_ref, wh_ref, b_ref, o_ref):
        h = jnp.zeros((BN, H_), jnp.float32)
        c = jnp.zeros((BN, H_), jnp.float32)
        for s_ref in (s0, s1, s2, s3):
            g = (jnp.dot(s_ref[...], wi_ref[...],
                         preferred_element_type=jnp.float32, precision=_PREC)
                 + jnp.dot(h, wh_ref[...],
                           preferred_element_type=jnp.float32, precision=_PREC)
                 + b_ref[...])
            i = jax.nn.sigmoid(g[:, :H_])
            f = jax.nn.sigmoid(g[:, H_:2 * H_])
            gg = jnp.tanh(g[:, 2 * H_:3 * H_])
            o = jax.nn.sigmoid(g[:, 3 * H_:])
            c = f * c + i * gg
            h = o * jnp.tanh(c)
        o_ref[...] = h
    nblock = pl.BlockSpec((BN, H_), lambda i: (i, 0))
    return pl.pallas_call(
        lstm_body,
        grid=(N_ // BN,),
        in_specs=[nblock, nblock, nblock, nblock,
                  pl.BlockSpec((H_, 4 * H_), lambda i: (0, 0)),
                  pl.BlockSpec((H_, 4 * H_), lambda i: (0, 0)),
                  pl.BlockSpec((1, 4 * H_), lambda i: (0, 0))],
        out_specs=nblock,
        out_shape=jax.ShapeDtypeStruct((N_, H_), jnp.float32),
    )(*seq, wihT, whhT, bias2d)


def kernel(x, edge_indexs, tconv_w, tconv_b, w1l, b1l, w1r, w2l, b2l, w2r,
           wih, whh, bih, bhh):
    wks = jnp.transpose(tconv_w, (2, 1, 0))
    enh = _causal_conv(x, wks, tconv_b[None, :])
    w1lT, w1rT = w1l.T, w1r.T
    w2lT, w2rT = w2l.T, w2r.T
    outs = []
    for t in range(T_):
        xt = enh[t]
        src_e = edge_indexs[t, 0]
        dst_e = edge_indexs[t, 1]
        agg1, cnt_flat = _segsum_sc(xt, src_e, dst_e, with_count=True)
        cnt = cnt_flat.reshape(NCORE, N_, 1)
        h1 = _sage_dense(agg1, cnt, xt, w1lT, b1l[None, :], w1rT, relu=True)
        (agg2,) = _segsum_sc(h1, src_e, dst_e, with_count=False)
        h2 = _sage_dense(agg2, cnt, h1, w2lT, b2l[None, :], w2rT, relu=False)
        outs.append(h2)
    return _lstm(outs, wih.T, whh.T, (bih + bhh)[None, :])


# speedup vs baseline: 6.7805x; 1.0019x over previous
import functools

import jax
import jax.numpy as jnp
from jax import lax
from jax.experimental import pallas as pl
from jax.experimental.pallas import tpu as pltpu
from jax.experimental.pallas import tpu_sc as plsc

N_, C_, H_, T_, E_, K_ = 10000, 128, 128, 4, 320000, 3
NCORE, NSUB, LANE = 2, 16, 16
NTILE = NCORE * NSUB
EPT = E_ // NTILE
CH = 80
NCHUNK = EPT // CH
NSC = CH // LANE
ROWB = 624
BN = 1000
_PREC = jax.lax.Precision.HIGHEST


def _segsum_sc(feats, src_e, dst_e, with_count):
    mesh = plsc.VectorSubcoreMesh(core_axis_name="c", subcore_axis_name="s",
                                  num_cores=NCORE, num_subcores=NSUB)
    out_type = [jax.ShapeDtypeStruct((NCORE, N_, C_), jnp.float32)]
    scratch = [
        pltpu.VMEM_SHARED((N_, C_), jnp.float32),
        pltpu.VMEM((EPT,), jnp.int32),
        pltpu.VMEM((EPT,), jnp.int32),
        pltpu.VMEM((CH, C_), jnp.float32),
        pltpu.VMEM((CH, C_), jnp.float32),
        pltpu.SemaphoreType.DMA,
        pltpu.SemaphoreType.DMA,
        pltpu.SemaphoreType.DMA,
        pltpu.SemaphoreType.DMA,
        pltpu.SemaphoreType.DMA,
        pltpu.SemaphoreType.DMA,
    ]
    if with_count:
        out_type.append(jax.ShapeDtypeStruct((NCORE * N_,), jnp.float32))
        scratch += [
            pltpu.VMEM_SHARED((N_,), jnp.float32),
            pltpu.VMEM((CH,), jnp.float32),
            pltpu.VMEM((LANE,), jnp.float32),
            pltpu.VMEM((ROWB,), jnp.float32),
            pltpu.SemaphoreType.DMA,
            pltpu.SemaphoreType.DMA,
        ]

    def body(x_hbm, srce_hbm, dste_hbm, *rest):
        if with_count:
            (out_hbm, cnt_hbm, agg_s, src_v, dst_v, bufA, bufB,
             semA, semB, semZ, semI, semR, semO,
             cnt_s, zc, ones_v, cbuf, semC, semO2) = rest
        else:
            (out_hbm, agg_s, src_v, dst_v, bufA, bufB,
             semA, semB, semZ, semI, semR, semO) = rest
        cid = lax.axis_index("c")
        sid = lax.axis_index("s")
        w = cid * NSUB + sid
        z16 = jnp.zeros((LANE,), jnp.float32)

        def zrow(r, carry):
            for k in range(C_ // LANE):
                bufA[r, pl.ds(k * LANE, LANE)] = z16
            return carry
        lax.fori_loop(0, CH, zrow, 0)
        if with_count:
            for k in range(CH // LANE):
                zc[pl.ds(k * LANE, LANE)] = z16
            ones_v[pl.ds(0, LANE)] = jnp.full((LANE,), 1.0, jnp.float32)

        base = sid * ROWB
        descs = []
        for j in range(7):
            descs.append(pltpu.async_copy(
                bufA.at[pl.ds(0, CH)], agg_s.at[pl.ds(base + j * CH, CH)], semZ))
        descs.append(pltpu.async_copy(
            bufA.at[pl.ds(0, ROWB - 7 * CH)],
            agg_s.at[pl.ds(base + 7 * CH, ROWB - 7 * CH)], semZ))
        if with_count:
            descs.append(pltpu.async_copy(
                zc.at[pl.ds(0, CH)], cnt_s.at[pl.ds(base, CH)], semZ))
            for j in range(1, 7):
                descs.append(pltpu.async_copy(
                    zc.at[pl.ds(0, CH)], cnt_s.at[pl.ds(base + j * CH, CH)], semZ))
            descs.append(pltpu.async_copy(
                zc.at[pl.ds(0, ROWB - 7 * CH)],
                cnt_s.at[pl.ds(base + 7 * CH, ROWB - 7 * CH)], semZ))

        @pl.when(sid == NSUB - 1)
        def _zero_tail():
            tds = [pltpu.async_copy(
                bufA.at[pl.ds(0, N_ - NSUB * ROWB)],
                agg_s.at[pl.ds(NSUB * ROWB, N_ - NSUB * ROWB)], semZ)]
            if with_count:
                tds.append(pltpu.async_copy(
                    zc.at[pl.ds(0, N_ - NSUB * ROWB)],
                    cnt_s.at[pl.ds(NSUB * ROWB, N_ - NSUB * ROWB)], semZ))
            for d in tds:
                d.wait()
        descs.append(pltpu.async_copy(
            srce_hbm.at[pl.ds(w * EPT, EPT)], src_v, semI))
        descs.append(pltpu.async_copy(
            dste_hbm.at[pl.ds(w * EPT, EPT)], dst_v, semI))
        for d in descs:
            d.wait()
        plsc.subcore_barrier()

        def gather(c, buf, sem):
            for k in range(NSC):
                srcv = src_v[pl.ds(c * CH + k * LANE, LANE)]
                pltpu.async_copy(
                    x_hbm.at[srcv], buf.at[pl.ds(k * LANE, LANE)], sem)

        def wait_gather(buf, sem):
            pltpu.make_async_copy(x_hbm.at[pl.ds(0, CH)], buf, sem).wait()

        def scatter_add(c, buf):
            for k in range(NSC):
                dstv = dst_v[pl.ds(c * CH + k * LANE, LANE)]
                pltpu.async_copy(
                    buf.at[pl.ds(k * LANE, LANE)], agg_s.at[dstv], semR,
                    add=True)
                if with_count:
                    pltpu.async_copy(ones_v, cnt_s.at[dstv], semC, add=True)
            pltpu.make_async_copy(x_hbm.at[pl.ds(0, CH)], buf, semR).wait()
            if with_count:
                pltpu.make_async_copy(
                    x_hbm.at[0, pl.ds(0, CH)], zc, semC).wait()

        gather(0, bufA, semA)
        gather(1, bufB, semB)

        def loop_body(i, carry):
            cA = 2 * i
            wait_gather(bufA, semA)
            scatter_add(cA, bufA)
            gather(cA + 2, bufA, semA)
            cB = 2 * i + 1
            wait_gather(bufB, semB)
            scatter_add(cB, bufB)
            cB2 = jnp.minimum(cB + 2, NCHUNK - 1)
            gather(cB2, bufB, semB)
            return carry
        lax.fori_loop(0, (NCHUNK - 1) // 2, loop_body, 0)

        cL = NCHUNK - 1
        wait_gather(bufA, semA)
        scatter_add(cL, bufA)
        wait_gather(bufB, semB)

        plsc.subcore_barrier()

        outd = [pltpu.async_copy(
            agg_s.at[pl.ds(base, ROWB)],
            out_hbm.at[cid, pl.ds(base, ROWB)], semO)]
        if with_count:
            pltpu.sync_copy(cnt_s.at[pl.ds(base, ROWB)], cbuf)
            outd.append(pltpu.async_copy(
                cbuf, cnt_hbm.at[pl.ds(cid * N_ + base, ROWB)], semO2))
        for d in outd:
            d.wait()

        @pl.when(sid == NSUB - 1)
        def _read_tail():
            tds = [pltpu.async_copy(
                agg_s.at[pl.ds(NSUB * ROWB, N_ - NSUB * ROWB)],
                out_hbm.at[cid, pl.ds(NSUB * ROWB, N_ - NSUB * ROWB)], semO)]
            if with_count:
                pltpu.sync_copy(cnt_s.at[pl.ds(NSUB * ROWB, N_ - NSUB * ROWB)],
                                cbuf.at[pl.ds(0, N_ - NSUB * ROWB)])
                tds.append(pltpu.async_copy(
                    cbuf.at[pl.ds(0, N_ - NSUB * ROWB)],
                    cnt_hbm.at[pl.ds(cid * N_ + NSUB * ROWB, N_ - NSUB * ROWB)],
                    semO2))
            for d in tds:
                d.wait()

    f = pl.kernel(body, out_type=tuple(out_type), mesh=mesh,
                  scratch_types=tuple(scratch))
    return f(feats, src_e, dst_e)


def _causal_conv(x, wks, b2d):
    def conv_body(x_ref, w_ref, b_ref, o_ref):
        for t in range(T_):
            s = None
            for k in range(K_):
                tt = t + k - (K_ - 1)
                if tt < 0:
                    continue
                m = jnp.dot(x_ref[tt], w_ref[k],
                            preferred_element_type=jnp.float32,
                            precision=_PREC)
                s = m if s is None else s + m
            o_ref[t] = s + b_ref[...]
    return pl.pallas_call(
        conv_body,
        grid=(N_ // BN,),
        in_specs=[
            pl.BlockSpec((T_, BN, C_), lambda i: (0, i, 0)),
            pl.BlockSpec((K_, C_, C_), lambda i: (0, 0, 0)),
            pl.BlockSpec((1, C_), lambda i: (0, 0)),
        ],
        out_specs=pl.BlockSpec((T_, BN, C_), lambda i: (0, i, 0)),
        out_shape=jax.ShapeDtypeStruct((T_, N_, C_), jnp.float32),
    )(x, wks, b2d)


def _sage_dense(agg2, cnt2, xin, wlT, bl2d, wrT, relu):
    def dense_body(a_ref, c_ref, x_ref, wl_ref, b_ref, wr_ref, o_ref):
        agg = a_ref[0] + a_ref[1]
        cnt = c_ref[0, :, 0] + c_ref[1, :, 0]
        mean = agg * (1.0 / jnp.maximum(cnt, 1.0))[:, None]
        y = (jnp.dot(mean, wl_ref[...], preferred_element_type=jnp.float32,
                     precision=_PREC)
             + b_ref[...]
             + jnp.dot(x_ref[...], wr_ref[...],
                       preferred_element_type=jnp.float32, precision=_PREC))
        o_ref[...] = jnp.maximum(y, 0.0) if relu else y
    return pl.pallas_call(
        dense_body,
        grid=(N_ // BN,),
        in_specs=[
            pl.BlockSpec((NCORE, BN, C_), lambda i: (0, i, 0)),
            pl.BlockSpec((NCORE, BN, 1), lambda i: (0, i, 0)),
            pl.BlockSpec((BN, C_), lambda i: (i, 0)),
            pl.BlockSpec((C_, H_), lambda i: (0, 0)),
            pl.BlockSpec((1, H_), lambda i: (0, 0)),
            pl.BlockSpec((C_, H_), lambda i: (0, 0)),
        ],
        out_specs=pl.BlockSpec((BN, H_), lambda i: (i, 0)),
        out_shape=jax.ShapeDtypeStruct((N_, H_), jnp.float32),
    )(agg2, cnt2, xin, wlT, bl2d, wrT)


def _lstm(seq, wihT, whhT, bias2d):
    def lstm_body(s0, s1, s2, s3, wi_ref, wh_ref, b_ref, o_ref):
        h = jnp.zeros((BN, H_), jnp.float32)
        c = jnp.zeros((BN, H_), jnp.float32)
        for s_ref in (s0, s1, s2, s3):
            g = (jnp.dot(s_ref[...], wi_ref[...],
                         preferred_element_type=jnp.float32, precision=_PREC)
                 + jnp.dot(h, wh_ref[...],
                           preferred_element_type=jnp.float32, precision=_PREC)
                 + b_ref[...])
            i = jax.nn.sigmoid(g[:, :H_])
            f = jax.nn.sigmoid(g[:, H_:2 * H_])
            gg = jnp.tanh(g[:, 2 * H_:3 * H_])
            o = jax.nn.sigmoid(g[:, 3 * H_:])
            c = f * c + i * gg
            h = o * jnp.tanh(c)
        o_ref[...] = h
    nblock = pl.BlockSpec((BN, H_), lambda i: (i, 0))
    return pl.pallas_call(
        lstm_body,
        grid=(N_ // BN,),
        in_specs=[nblock, nblock, nblock, nblock,
                  pl.BlockSpec((H_, 4 * H_), lambda i: (0, 0)),
                  pl.BlockSpec((H_, 4 * H_), lambda i: (0, 0)),
                  pl.BlockSpec((1, 4 * H_), lambda i: (0, 0))],
        out_specs=nblock,
        out_shape=jax.ShapeDtypeStruct((N_, H_), jnp.float32),
    )(*seq, wihT, whhT, bias2d)


def kernel(x, edge_indexs, tconv_w, tconv_b, w1l, b1l, w1r, w2l, b2l, w2r,
           wih, whh, bih, bhh):
    wks = jnp.transpose(tconv_w, (2, 1, 0))
    enh = _causal_conv(x, wks, tconv_b[None, :])
    w1lT, w1rT = w1l.T, w1r.T
    w2lT, w2rT = w2l.T, w2r.T
    outs = []
    for t in range(T_):
        xt = enh[t]
        src_e = edge_indexs[t, 0]
        dst_e = edge_indexs[t, 1]
        agg1, cnt_flat = _segsum_sc(xt, src_e, dst_e, with_count=True)
        cnt = cnt_flat.reshape(NCORE, N_, 1)
        h1 = _sage_dense(agg1, cnt, xt, w1lT, b1l[None, :], w1rT, relu=True)
        (agg2,) = _segsum_sc(h1, src_e, dst_e, with_count=False)
        h2 = _sage_dense(agg2, cnt, h1, w2lT, b2l[None, :], w2rT, relu=False)
        outs.append(h2)
    return _lstm(outs, wih.T, whh.T, (bih + bhh)[None, :])


# speedup vs baseline: 7.8303x; 1.1548x over previous
import functools

import jax
import jax.numpy as jnp
from jax import lax
from jax.experimental import pallas as pl
from jax.experimental.pallas import tpu as pltpu
from jax.experimental.pallas import tpu_sc as plsc

N_, C_, H_, T_, E_, K_ = 10000, 128, 128, 4, 320000, 3
NCORE, NSUB, LANE = 2, 16, 16
NTILE = NCORE * NSUB
EPT = E_ // NTILE
CH = 80
NCHUNK = EPT // CH
NSC = CH // LANE
ROWB = 624
BN = 1000
_PREC = jax.lax.Precision.DEFAULT


def _segsum_sc(feats, src_e, dst_e, with_count):
    mesh = plsc.VectorSubcoreMesh(core_axis_name="c", subcore_axis_name="s",
                                  num_cores=NCORE, num_subcores=NSUB)
    out_type = [jax.ShapeDtypeStruct((NCORE, N_, C_), jnp.float32)]
    scratch = [
        pltpu.VMEM_SHARED((N_, C_), jnp.float32),
        pltpu.VMEM((EPT,), jnp.int32),
        pltpu.VMEM((EPT,), jnp.int32),
        pltpu.VMEM((CH, C_), jnp.float32),
        pltpu.VMEM((CH, C_), jnp.float32),
        pltpu.SemaphoreType.DMA,
        pltpu.SemaphoreType.DMA,
        pltpu.SemaphoreType.DMA,
        pltpu.SemaphoreType.DMA,
        pltpu.SemaphoreType.DMA,
        pltpu.SemaphoreType.DMA,
    ]
    if with_count:
        out_type.append(jax.ShapeDtypeStruct((NCORE * N_,), jnp.float32))
        scratch += [
            pltpu.VMEM_SHARED((N_,), jnp.float32),
            pltpu.VMEM((CH,), jnp.float32),
            pltpu.VMEM((LANE,), jnp.float32),
            pltpu.VMEM((ROWB,), jnp.float32),
            pltpu.SemaphoreType.DMA,
            pltpu.SemaphoreType.DMA,
        ]

    def body(x_hbm, srce_hbm, dste_hbm, *rest):
        if with_count:
            (out_hbm, cnt_hbm, agg_s, src_v, dst_v, bufA, bufB,
             semA, semB, semZ, semI, semR, semO,
             cnt_s, zc, ones_v, cbuf, semC, semO2) = rest
        else:
            (out_hbm, agg_s, src_v, dst_v, bufA, bufB,
             semA, semB, semZ, semI, semR, semO) = rest
        cid = lax.axis_index("c")
        sid = lax.axis_index("s")
        w = cid * NSUB + sid
        z16 = jnp.zeros((LANE,), jnp.float32)

        def zrow(r, carry):
            for k in range(C_ // LANE):
                bufA[r, pl.ds(k * LANE, LANE)] = z16
            return carry
        lax.fori_loop(0, CH, zrow, 0)
        if with_count:
            for k in range(CH // LANE):
                zc[pl.ds(k * LANE, LANE)] = z16
            ones_v[pl.ds(0, LANE)] = jnp.full((LANE,), 1.0, jnp.float32)

        base = sid * ROWB
        descs = []
        for j in range(7):
            descs.append(pltpu.async_copy(
                bufA.at[pl.ds(0, CH)], agg_s.at[pl.ds(base + j * CH, CH)], semZ))
        descs.append(pltpu.async_copy(
            bufA.at[pl.ds(0, ROWB - 7 * CH)],
            agg_s.at[pl.ds(base + 7 * CH, ROWB - 7 * CH)], semZ))
        if with_count:
            descs.append(pltpu.async_copy(
                zc.at[pl.ds(0, CH)], cnt_s.at[pl.ds(base, CH)], semZ))
            for j in range(1, 7):
                descs.append(pltpu.async_copy(
                    zc.at[pl.ds(0, CH)], cnt_s.at[pl.ds(base + j * CH, CH)], semZ))
            descs.append(pltpu.async_copy(
                zc.at[pl.ds(0, ROWB - 7 * CH)],
                cnt_s.at[pl.ds(base + 7 * CH, ROWB - 7 * CH)], semZ))

        @pl.when(sid == NSUB - 1)
        def _zero_tail():
            tds = [pltpu.async_copy(
                bufA.at[pl.ds(0, N_ - NSUB * ROWB)],
                agg_s.at[pl.ds(NSUB * ROWB, N_ - NSUB * ROWB)], semZ)]
            if with_count:
                tds.append(pltpu.async_copy(
                    zc.at[pl.ds(0, N_ - NSUB * ROWB)],
                    cnt_s.at[pl.ds(NSUB * ROWB, N_ - NSUB * ROWB)], semZ))
            for d in tds:
                d.wait()
        descs.append(pltpu.async_copy(
            srce_hbm.at[pl.ds(w * EPT, EPT)], src_v, semI))
        descs.append(pltpu.async_copy(
            dste_hbm.at[pl.ds(w * EPT, EPT)], dst_v, semI))
        for d in descs:
            d.wait()
        plsc.subcore_barrier()

        def gather(c, buf, sem):
            for k in range(NSC):
                srcv = src_v[pl.ds(c * CH + k * LANE, LANE)]
                pltpu.async_copy(
                    x_hbm.at[srcv], buf.at[pl.ds(k * LANE, LANE)], sem)

        def wait_gather(buf, sem):
            pltpu.make_async_copy(x_hbm.at[pl.ds(0, CH)], buf, sem).wait()

        def scatter_add(c, buf):
            for k in range(NSC):
                dstv = dst_v[pl.ds(c * CH + k * LANE, LANE)]
                pltpu.async_copy(
                    buf.at[pl.ds(k * LANE, LANE)], agg_s.at[dstv], semR,
                    add=True)
                if with_count:
                    pltpu.async_copy(ones_v, cnt_s.at[dstv], semC, add=True)
            pltpu.make_async_copy(x_hbm.at[pl.ds(0, CH)], buf, semR).wait()
            if with_count:
                pltpu.make_async_copy(
                    x_hbm.at[0, pl.ds(0, CH)], zc, semC).wait()

        gather(0, bufA, semA)
        gather(1, bufB, semB)

        def loop_body(i, carry):
            cA = 2 * i
            wait_gather(bufA, semA)
            scatter_add(cA, bufA)
            gather(cA + 2, bufA, semA)
            cB = 2 * i + 1
            wait_gather(bufB, semB)
            scatter_add(cB, bufB)
            cB2 = jnp.minimum(cB + 2, NCHUNK - 1)
            gather(cB2, bufB, semB)
            return carry
        lax.fori_loop(0, (NCHUNK - 1) // 2, loop_body, 0)

        cL = NCHUNK - 1
        wait_gather(bufA, semA)
        scatter_add(cL, bufA)
        wait_gather(bufB, semB)

        plsc.subcore_barrier()

        outd = [pltpu.async_copy(
            agg_s.at[pl.ds(base, ROWB)],
            out_hbm.at[cid, pl.ds(base, ROWB)], semO)]
        if with_count:
            pltpu.sync_copy(cnt_s.at[pl.ds(base, ROWB)], cbuf)
            outd.append(pltpu.async_copy(
                cbuf, cnt_hbm.at[pl.ds(cid * N_ + base, ROWB)], semO2))
        for d in outd:
            d.wait()

        @pl.when(sid == NSUB - 1)
        def _read_tail():
            tds = [pltpu.async_copy(
                agg_s.at[pl.ds(NSUB * ROWB, N_ - NSUB * ROWB)],
                out_hbm.at[cid, pl.ds(NSUB * ROWB, N_ - NSUB * ROWB)], semO)]
            if with_count:
                pltpu.sync_copy(cnt_s.at[pl.ds(NSUB * ROWB, N_ - NSUB * ROWB)],
                                cbuf.at[pl.ds(0, N_ - NSUB * ROWB)])
                tds.append(pltpu.async_copy(
                    cbuf.at[pl.ds(0, N_ - NSUB * ROWB)],
                    cnt_hbm.at[pl.ds(cid * N_ + NSUB * ROWB, N_ - NSUB * ROWB)],
                    semO2))
            for d in tds:
                d.wait()

    f = pl.kernel(body, out_type=tuple(out_type), mesh=mesh,
                  scratch_types=tuple(scratch))
    return f(feats, src_e, dst_e)


def _causal_conv(x, wks, b2d):
    def conv_body(x_ref, w_ref, b_ref, o_ref):
        for t in range(T_):
            s = None
            for k in range(K_):
                tt = t + k - (K_ - 1)
                if tt < 0:
                    continue
                m = jnp.dot(x_ref[tt], w_ref[k],
                            preferred_element_type=jnp.float32,
                            precision=_PREC)
                s = m if s is None else s + m
            o_ref[t] = s + b_ref[...]
    return pl.pallas_call(
        conv_body,
        grid=(N_ // BN,),
        in_specs=[
            pl.BlockSpec((T_, BN, C_), lambda i: (0, i, 0)),
            pl.BlockSpec((K_, C_, C_), lambda i: (0, 0, 0)),
            pl.BlockSpec((1, C_), lambda i: (0, 0)),
        ],
        out_specs=pl.BlockSpec((T_, BN, C_), lambda i: (0, i, 0)),
        out_shape=jax.ShapeDtypeStruct((T_, N_, C_), jnp.float32),
    )(x, wks, b2d)


def _sage_dense(agg2, cnt2, xin, wlT, bl2d, wrT, relu):
    def dense_body(a_ref, c_ref, x_ref, wl_ref, b_ref, wr_ref, o_ref):
        agg = a_ref[0] + a_ref[1]
        cnt = c_ref[0, :, 0] + c_ref[1, :, 0]
        mean = agg * (1.0 / jnp.maximum(cnt, 1.0))[:, None]
        y = (jnp.dot(mean, wl_ref[...], preferred_element_type=jnp.float32,
                     precision=_PREC)
             + b_ref[...]
             + jnp.dot(x_ref[...], wr_ref[...],
                       preferred_element_type=jnp.float32, precision=_PREC))
        o_ref[...] = jnp.maximum(y, 0.0) if relu else y
    return pl.pallas_call(
        dense_body,
        grid=(N_ // BN,),
        in_specs=[
            pl.BlockSpec((NCORE, BN, C_), lambda i: (0, i, 0)),
            pl.BlockSpec((NCORE, BN, 1), lambda i: (0, i, 0)),
            pl.BlockSpec((BN, C_), lambda i: (i, 0)),
            pl.BlockSpec((C_, H_), lambda i: (0, 0)),
            pl.BlockSpec((1, H_), lambda i: (0, 0)),
            pl.BlockSpec((C_, H_), lambda i: (0, 0)),
        ],
        out_specs=pl.BlockSpec((BN, H_), lambda i: (i, 0)),
        out_shape=jax.ShapeDtypeStruct((N_, H_), jnp.float32),
    )(agg2, cnt2, xin, wlT, bl2d, wrT)


def _lstm(seq, wihT, whhT, bias2d):
    def lstm_body(s0, s1, s2, s3, wi_ref, wh_ref, b_ref, o_ref):
        h = jnp.zeros((BN, H_), jnp.float32)
        c = jnp.zeros((BN, H_), jnp.float32)
        for s_ref in (s0, s1, s2, s3):
            g = (jnp.dot(s_ref[...], wi_ref[...],
                         preferred_element_type=jnp.float32, precision=_PREC)
                 + jnp.dot(h, wh_ref[...],
                           preferred_element_type=jnp.float32, precision=_PREC)
                 + b_ref[...])
            i = jax.nn.sigmoid(g[:, :H_])
            f = jax.nn.sigmoid(g[:, H_:2 * H_])
            gg = jnp.tanh(g[:, 2 * H_:3 * H_])
            o = jax.nn.sigmoid(g[:, 3 * H_:])
            c = f * c + i * gg
            h = o * jnp.tanh(c)
        o_ref[...] = h
    nblock = pl.BlockSpec((BN, H_), lambda i: (i, 0))
    return pl.pallas_call(
        lstm_body,
        grid=(N_ // BN,),
        in_specs=[nblock, nblock, nblock, nblock,
                  pl.BlockSpec((H_, 4 * H_), lambda i: (0, 0)),
                  pl.BlockSpec((H_, 4 * H_), lambda i: (0, 0)),
                  pl.BlockSpec((1, 4 * H_), lambda i: (0, 0))],
        out_specs=nblock,
        out_shape=jax.ShapeDtypeStruct((N_, H_), jnp.float32),
    )(*seq, wihT, whhT, bias2d)


def kernel(x, edge_indexs, tconv_w, tconv_b, w1l, b1l, w1r, w2l, b2l, w2r,
           wih, whh, bih, bhh):
    wks = jnp.transpose(tconv_w, (2, 1, 0))
    enh = _causal_conv(x, wks, tconv_b[None, :])
    w1lT, w1rT = w1l.T, w1r.T
    w2lT, w2rT = w2l.T, w2r.T
    outs = []
    for t in range(T_):
        xt = enh[t]
        src_e = edge_indexs[t, 0]
        dst_e = edge_indexs[t, 1]
        agg1, cnt_flat = _segsum_sc(xt, src_e, dst_e, with_count=True)
        cnt = cnt_flat.reshape(NCORE, N_, 1)
        h1 = _sage_dense(agg1, cnt, xt, w1lT, b1l[None, :], w1rT, relu=True)
        (agg2,) = _segsum_sc(h1, src_e, dst_e, with_count=False)
        h2 = _sage_dense(agg2, cnt, h1, w2lT, b2l[None, :], w2rT, relu=False)
        outs.append(h2)
    return _lstm(outs, wih.T, whh.T, (bih + bhh)[None, :])


# speedup vs baseline: 7.8314x; 1.0001x over previous
import jax
import jax.numpy as jnp
from jax import lax
from jax.experimental import pallas as pl
from jax.experimental.pallas import tpu as pltpu
from jax.experimental.pallas import tpu_sc as plsc

N_, C_, H_, T_, E_, K_ = 10000, 128, 128, 4, 320000, 3
NCORE, NSUB, LANE = 2, 16, 16
NTILE = NCORE * NSUB
EPT = E_ // NTILE
CH = 80
NCHUNK = EPT // CH
NSC = CH // LANE
ROWB = 624
BN = 1000
_PREC = jax.lax.Precision.DEFAULT


def _segsum_sc(feats, src_e, dst_e, with_count):
    mesh = plsc.VectorSubcoreMesh(core_axis_name="c", subcore_axis_name="s",
                                  num_cores=NCORE, num_subcores=NSUB)
    out_type = [jax.ShapeDtypeStruct((NCORE, N_, C_), jnp.float32)]
    scratch = [
        pltpu.VMEM_SHARED((N_, C_), jnp.float32),
        pltpu.VMEM((EPT,), jnp.int32),
        pltpu.VMEM((EPT,), jnp.int32),
        pltpu.VMEM((CH, C_), jnp.float32),
        pltpu.VMEM((CH, C_), jnp.float32),
        pltpu.SemaphoreType.DMA,
        pltpu.SemaphoreType.DMA,
        pltpu.SemaphoreType.DMA,
        pltpu.SemaphoreType.DMA,
        pltpu.SemaphoreType.DMA,
        pltpu.SemaphoreType.DMA,
    ]
    if with_count:
        out_type.append(jax.ShapeDtypeStruct((NCORE * N_,), jnp.float32))
        scratch += [
            pltpu.VMEM_SHARED((N_,), jnp.float32),
            pltpu.VMEM((CH,), jnp.float32),
            pltpu.VMEM((LANE,), jnp.float32),
            pltpu.VMEM((ROWB,), jnp.float32),
            pltpu.SemaphoreType.DMA,
            pltpu.SemaphoreType.DMA,
        ]

    def body(x_hbm, srce_hbm, dste_hbm, *rest):
        if with_count:
            (out_hbm, cnt_hbm, agg_s, src_v, dst_v, bufA, bufB,
             semA, semB, semZ, semI, semR, semO,
             cnt_s, zc, ones_v, cbuf, semC, semO2) = rest
        else:
            (out_hbm, agg_s, src_v, dst_v, bufA, bufB,
             semA, semB, semZ, semI, semR, semO) = rest
        cid = lax.axis_index("c")
        sid = lax.axis_index("s")
        w = cid * NSUB + sid
        z16 = jnp.zeros((LANE,), jnp.float32)

        def zrow(r, carry):
            for k in range(C_ // LANE):
                bufA[r, pl.ds(k * LANE, LANE)] = z16
            return carry
        lax.fori_loop(0, CH, zrow, 0)
        if with_count:
            for k in range(CH // LANE):
                zc[pl.ds(k * LANE, LANE)] = z16
            ones_v[pl.ds(0, LANE)] = jnp.full((LANE,), 1.0, jnp.float32)

        base = sid * ROWB
        descs = []
        for j in range(7):
            descs.append(pltpu.async_copy(
                bufA.at[pl.ds(0, CH)], agg_s.at[pl.ds(base + j * CH, CH)], semZ))
        descs.append(pltpu.async_copy(
            bufA.at[pl.ds(0, ROWB - 7 * CH)],
            agg_s.at[pl.ds(base + 7 * CH, ROWB - 7 * CH)], semZ))
        if with_count:
            descs.append(pltpu.async_copy(
                zc.at[pl.ds(0, CH)], cnt_s.at[pl.ds(base, CH)], semZ))
            for j in range(1, 7):
                descs.append(pltpu.async_copy(
                    zc.at[pl.ds(0, CH)], cnt_s.at[pl.ds(base + j * CH, CH)], semZ))
            descs.append(pltpu.async_copy(
                zc.at[pl.ds(0, ROWB - 7 * CH)],
                cnt_s.at[pl.ds(base + 7 * CH, ROWB - 7 * CH)], semZ))

        @pl.when(sid == NSUB - 1)
        def _zero_tail():
            tds = [pltpu.async_copy(
                bufA.at[pl.ds(0, N_ - NSUB * ROWB)],
                agg_s.at[pl.ds(NSUB * ROWB, N_ - NSUB * ROWB)], semZ)]
            if with_count:
                tds.append(pltpu.async_copy(
                    zc.at[pl.ds(0, N_ - NSUB * ROWB)],
                    cnt_s.at[pl.ds(NSUB * ROWB, N_ - NSUB * ROWB)], semZ))
            for d in tds:
                d.wait()
        descs.append(pltpu.async_copy(
            srce_hbm.at[pl.ds(w * EPT, EPT)], src_v, semI))
        descs.append(pltpu.async_copy(
            dste_hbm.at[pl.ds(w * EPT, EPT)], dst_v, semI))
        for d in descs:
            d.wait()
        plsc.subcore_barrier()

        def gather(c, buf, sem):
            for k in range(NSC):
                srcv = src_v[pl.ds(c * CH + k * LANE, LANE)]
                pltpu.async_copy(
                    x_hbm.at[srcv], buf.at[pl.ds(k * LANE, LANE)], sem)

        def wait_gather(buf, sem):
            pltpu.make_async_copy(x_hbm.at[pl.ds(0, CH)], buf, sem).wait()

        def scatter_add(c, buf):
            for k in range(NSC):
                dstv = dst_v[pl.ds(c * CH + k * LANE, LANE)]
                pltpu.async_copy(
                    buf.at[pl.ds(k * LANE, LANE)], agg_s.at[dstv], semR,
                    add=True)
                if with_count:
                    pltpu.async_copy(ones_v, cnt_s.at[dstv], semC, add=True)
            pltpu.make_async_copy(x_hbm.at[pl.ds(0, CH)], buf, semR).wait()
            if with_count:
                pltpu.make_async_copy(
                    x_hbm.at[0, pl.ds(0, CH)], zc, semC).wait()

        gather(0, bufA, semA)
        gather(1, bufB, semB)

        def loop_body(i, carry):
            cA = 2 * i
            wait_gather(bufA, semA)
            scatter_add(cA, bufA)
            gather(cA + 2, bufA, semA)
            cB = 2 * i + 1
            wait_gather(bufB, semB)
            scatter_add(cB, bufB)
            cB2 = jnp.minimum(cB + 2, NCHUNK - 1)
            gather(cB2, bufB, semB)
            return carry
        lax.fori_loop(0, (NCHUNK - 1) // 2, loop_body, 0)

        cL = NCHUNK - 1
        wait_gather(bufA, semA)
        scatter_add(cL, bufA)
        wait_gather(bufB, semB)

        plsc.subcore_barrier()

        outd = [pltpu.async_copy(
            agg_s.at[pl.ds(base, ROWB)],
            out_hbm.at[cid, pl.ds(base, ROWB)], semO)]
        if with_count:
            pltpu.sync_copy(cnt_s.at[pl.ds(base, ROWB)], cbuf)
            outd.append(pltpu.async_copy(
                cbuf, cnt_hbm.at[pl.ds(cid * N_ + base, ROWB)], semO2))
        for d in outd:
            d.wait()

        @pl.when(sid == NSUB - 1)
        def _read_tail():
            tds = [pltpu.async_copy(
                agg_s.at[pl.ds(NSUB * ROWB, N_ - NSUB * ROWB)],
                out_hbm.at[cid, pl.ds(NSUB * ROWB, N_ - NSUB * ROWB)], semO)]
            if with_count:
                pltpu.sync_copy(cnt_s.at[pl.ds(NSUB * ROWB, N_ - NSUB * ROWB)],
                                cbuf.at[pl.ds(0, N_ - NSUB * ROWB)])
                tds.append(pltpu.async_copy(
                    cbuf.at[pl.ds(0, N_ - NSUB * ROWB)],
                    cnt_hbm.at[pl.ds(cid * N_ + NSUB * ROWB, N_ - NSUB * ROWB)],
                    semO2))
            for d in tds:
                d.wait()

    f = pl.kernel(body, out_type=tuple(out_type), mesh=mesh,
                  scratch_types=tuple(scratch))
    return f(feats, src_e, dst_e)


def _causal_conv(x, wks, b2d):
    def conv_body(x_ref, w_ref, b_ref, o_ref):
        for t in range(T_):
            s = None
            for k in range(K_):
                tt = t + k - (K_ - 1)
                if tt < 0:
                    continue
                m = jnp.dot(x_ref[tt], w_ref[k],
                            preferred_element_type=jnp.float32,
                            precision=_PREC)
                s = m if s is None else s + m
            o_ref[t] = s + b_ref[...]
    return pl.pallas_call(
        conv_body,
        grid=(N_ // BN,),
        in_specs=[
            pl.BlockSpec((T_, BN, C_), lambda i: (0, i, 0)),
            pl.BlockSpec((K_, C_, C_), lambda i: (0, 0, 0)),
            pl.BlockSpec((1, C_), lambda i: (0, 0)),
        ],
        out_specs=pl.BlockSpec((T_, BN, C_), lambda i: (0, i, 0)),
        out_shape=jax.ShapeDtypeStruct((T_, N_, C_), jnp.float32),
    )(x, wks, b2d)


def _sage_dense(agg2, cnt2, xin, wlT, bl2d, wrT, relu):
    def dense_body(a_ref, c_ref, x_ref, wl_ref, b_ref, wr_ref, o_ref):
        agg = a_ref[0] + a_ref[1]
        cnt = c_ref[0, :, 0] + c_ref[1, :, 0]
        mean = agg * (1.0 / jnp.maximum(cnt, 1.0))[:, None]
        y = (jnp.dot(mean, wl_ref[...], preferred_element_type=jnp.float32,
                     precision=_PREC)
             + b_ref[...]
             + jnp.dot(x_ref[...], wr_ref[...],
                       preferred_element_type=jnp.float32, precision=_PREC))
        o_ref[...] = jnp.maximum(y, 0.0) if relu else y
    return pl.pallas_call(
        dense_body,
        grid=(N_ // BN,),
        in_specs=[
            pl.BlockSpec((NCORE, BN, C_), lambda i: (0, i, 0)),
            pl.BlockSpec((NCORE, BN, 1), lambda i: (0, i, 0)),
            pl.BlockSpec((BN, C_), lambda i: (i, 0)),
            pl.BlockSpec((C_, H_), lambda i: (0, 0)),
            pl.BlockSpec((1, H_), lambda i: (0, 0)),
            pl.BlockSpec((C_, H_), lambda i: (0, 0)),
        ],
        out_specs=pl.BlockSpec((BN, H_), lambda i: (i, 0)),
        out_shape=jax.ShapeDtypeStruct((N_, H_), jnp.float32),
    )(agg2, cnt2, xin, wlT, bl2d, wrT)


def _lstm(seq, wihT, whhT, bias2d):
    def lstm_body(s0, s1, s2, s3, wi_ref, wh_ref, b_ref, o_ref):
        h = jnp.zeros((BN, H_), jnp.float32)
        c = jnp.zeros((BN, H_), jnp.float32)
        for s_ref in (s0, s1, s2, s3):
            g = (jnp.dot(s_ref[...], wi_ref[...],
                         preferred_element_type=jnp.float32, precision=_PREC)
                 + jnp.dot(h, wh_ref[...],
                           preferred_element_type=jnp.float32, precision=_PREC)
                 + b_ref[...])
            i = jax.nn.sigmoid(g[:, :H_])
            f = jax.nn.sigmoid(g[:, H_:2 * H_])
            gg = jnp.tanh(g[:, 2 * H_:3 * H_])
            o = jax.nn.sigmoid(g[:, 3 * H_:])
            c = f * c + i * gg
            h = o * jnp.tanh(c)
        o_ref[...] = h
    nblock = pl.BlockSpec((BN, H_), lambda i: (i, 0))
    return pl.pallas_call(
        lstm_body,
        grid=(N_ // BN,),
        in_specs=[nblock, nblock, nblock, nblock,
                  pl.BlockSpec((H_, 4 * H_), lambda i: (0, 0)),
                  pl.BlockSpec((H_, 4 * H_), lambda i: (0, 0)),
                  pl.BlockSpec((1, 4 * H_), lambda i: (0, 0))],
        out_specs=nblock,
        out_shape=jax.ShapeDtypeStruct((N_, H_), jnp.float32),
    )(*seq, wihT, whhT, bias2d)


def kernel(x, edge_indexs, tconv_w, tconv_b, w1l, b1l, w1r, w2l, b2l, w2r,
           wih, whh, bih, bhh):
    wks = jnp.transpose(tconv_w, (2, 1, 0))
    enh = _causal_conv(x, wks, tconv_b[None, :])
    w1lT, w1rT = w1l.T, w1r.T
    w2lT, w2rT = w2l.T, w2r.T
    outs = []
    for t in range(T_):
        xt = enh[t]
        src_e = edge_indexs[t, 0]
        dst_e = edge_indexs[t, 1]
        agg1, cnt_flat = _segsum_sc(xt, src_e, dst_e, with_count=True)
        cnt = cnt_flat.reshape(NCORE, N_, 1)
        h1 = _sage_dense(agg1, cnt, xt, w1lT, b1l[None, :], w1rT, relu=True)
        (agg2,) = _segsum_sc(h1, src_e, dst_e, with_count=False)
        h2 = _sage_dense(agg2, cnt, h1, w2lT, b2l[None, :], w2rT, relu=False)
        outs.append(h2)
    return _lstm(outs, wih.T, whh.T, (bih + bhh)[None, :])
